# baseline (speedup 1.0000x reference)
import functools

import jax
import jax.numpy as jnp
from jax import lax
from jax.experimental import pallas as pl
from jax.experimental.pallas import tpu as pltpu
from jax.experimental.pallas import tpu_sc as plsc

N = 10000
E = 320000
D_HID = 128
D_OUT = 64

NC = 2
NS = 16
NTILE = NC * NS
NPAD = 10240
DEG_SLICE = NPAD // NS

EPAD = 327680
EPT = EPAD // NTILE

SEG_CHUNK = 128
SEG_G = EPT // SEG_CHUNK
NBUF = 2

DEG_CHUNK = 128
DEG_G = EPT // DEG_CHUNK

ROWS_PT = NPAD // NS
STG = 128

def _zero16():
  return jnp.zeros((16,), jnp.float32)


def _make_seg_sum(D, linear=False):
  HG = SEG_G // 2
  mesh = plsc.VectorSubcoreMesh(core_axis_name="c", subcore_axis_name="s")
  out_t = (jax.ShapeDtypeStruct((NPAD, D), jnp.float32),
           jax.ShapeDtypeStruct((NPAD, D), jnp.float32))
  scratch = ([pltpu.VMEM_SHARED((NPAD, D), jnp.float32),
              pltpu.VMEM((HG, SEG_CHUNK), jnp.int32),
              pltpu.VMEM((HG, SEG_CHUNK), jnp.int32),
              pltpu.VMEM((16, D), jnp.float32)]
             + [pltpu.VMEM((STG, D), jnp.float32) for _ in range(NBUF)]
             + [pltpu.SemaphoreType.DMA for _ in range(NBUF)])

  params = pltpu.CompilerParams(use_tc_tiling_on_sc=False) if linear else None

  @functools.partial(pl.kernel, out_type=out_t, mesh=mesh,
                     scratch_types=scratch, compiler_params=params)
  def seg(xs, src, dst, out0, out1, acc, srcv, dstv, zbuf,
          r0, r1, m0, m1):
    c = lax.axis_index("c")
    s = lax.axis_index("s")
    w = c * NS + s
    rows = (r0, r1)
    sems = (m0, m1)

    for r in range(16):
      for j in range(D // 16):
        zbuf[r, pl.ds(j * 16, 16)] = _zero16()

    def _zacc(i, carry):
      pltpu.sync_copy(zbuf, acc.at[pl.ds(s * ROWS_PT + i * 16, 16)])
      return carry
    lax.fori_loop(0, ROWS_PT // 16, _zacc, 0)
    plsc.subcore_barrier()

    def _fire(g, j):
      pltpu.async_copy(xs.at[srcv.at[g]], rows[j], sems[j])

    def _wait(g, j):
      pltpu.make_async_copy(xs.at[srcv.at[g]], rows[j], sems[j]).wait()

    for h in range(2):
      pltpu.sync_copy(src.at[pl.ds(w * SEG_G + h * HG, HG)], srcv)
      pltpu.sync_copy(dst.at[pl.ds(w * SEG_G + h * HG, HG)], dstv)

      for j in range(NBUF):
        _fire(j, j)

      def _group(i, carry):
        for j in range(NBUF):
          g = i * NBUF + j
          _wait(g, j)
          pltpu.sync_copy(rows[j], acc.at[dstv.at[g]], add=True)

          @pl.when(g + NBUF < HG)
          def _():
            _fire(g + NBUF, j)
        return carry
      lax.fori_loop(0, HG // NBUF, _group, 0)

    plsc.subcore_barrier()

    for k in range(ROWS_PT // STG):
      base = s * ROWS_PT + k * STG
      pltpu.sync_copy(acc.at[pl.ds(base, STG)], rows[0])

      @pl.when(c == 0)
      def _():
        pltpu.sync_copy(rows[0], out0.at[pl.ds(base, STG)])

      @pl.when(c == 1)
      def _():
        pltpu.sync_copy(rows[0], out1.at[pl.ds(base, STG)])

  return seg


def _make_deg():
  mesh = plsc.VectorSubcoreMesh(core_axis_name="c", subcore_axis_name="s")
  out_t = (jax.ShapeDtypeStruct((NPAD,), jnp.float32),
           jax.ShapeDtypeStruct((NPAD,), jnp.float32))
  scratch = ([pltpu.VMEM_SHARED((NPAD,), jnp.float32),
              pltpu.VMEM((DEG_G, DEG_CHUNK), jnp.int32),
              pltpu.VMEM((DEG_CHUNK,), jnp.float32)]
             + [pltpu.SemaphoreType.DMA for _ in range(4)])

  @functools.partial(pl.kernel, out_type=out_t, mesh=mesh,
                     scratch_types=scratch)
  def deg(dst, ones, zeros, out0, out1, acc, dstv, ones_v, m0, m1, m2, m3):
    c = lax.axis_index("c")
    s = lax.axis_index("s")
    w = c * NS + s
    sems = (m0, m1, m2, m3)

    pltpu.sync_copy(ones, ones_v)
    pltpu.sync_copy(dst.at[pl.ds(w * DEG_G, DEG_G)], dstv)
    pltpu.sync_copy(zeros.at[pl.ds(s * DEG_SLICE, DEG_SLICE)],
                    acc.at[pl.ds(s * DEG_SLICE, DEG_SLICE)])
    plsc.subcore_barrier()

    def _fire(g, j):
      pltpu.async_copy(ones_v, acc.at[dstv.at[g]], sems[j], add=True)

    def _wait(g, j):
      pltpu.make_async_copy(ones_v, acc.at[dstv.at[g]], sems[j]).wait()

    for j in range(4):
      _fire(j, j)

    def _group(i, carry):
      for j in range(4):
        g = i * 4 + j
        _wait(g, j)

        @pl.when(g + 4 < DEG_G)
        def _():
          _fire(g + 4, j)
      return carry
    lax.fori_loop(0, DEG_G // 4, _group, 0)

    plsc.subcore_barrier()

    @pl.when(c == 0)
    def _():
      pltpu.sync_copy(acc.at[pl.ds(s * DEG_SLICE, DEG_SLICE)],
                      out0.at[pl.ds(s * DEG_SLICE, DEG_SLICE)])

    @pl.when(c == 1)
    def _():
      pltpu.sync_copy(acc.at[pl.ds(s * DEG_SLICE, DEG_SLICE)],
                      out1.at[pl.ds(s * DEG_SLICE, DEG_SLICE)])

  return deg


_seg128 = _make_seg_sum(D_HID)
_seg64 = _make_seg_sum(D_OUT, linear=True)
_deg = _make_deg()

_R = 2048


def _row_spec(d):
  return pl.BlockSpec((_R, d), lambda i: (i, 0))


def _full_spec(shape):
  return pl.BlockSpec(shape, lambda i: (0, 0))


def _tc_prep(d0, d1, x):
  def body(d0_ref, d1_ref, x_ref, xs_ref, dinv_ref):
    deg = jnp.maximum(d0_ref[...] + d1_ref[...], 1.0)
    dinv = lax.rsqrt(deg)
    dinv_ref[...] = dinv
    xs_ref[...] = x_ref[...] * dinv

  return pl.pallas_call(
      body,
      grid=(NPAD // _R,),
      in_specs=[_row_spec(1), _row_spec(1), _row_spec(D_HID)],
      out_specs=[_row_spec(D_HID), _row_spec(1)],
      out_shape=[jax.ShapeDtypeStruct((NPAD, D_HID), jnp.float32),
                 jax.ShapeDtypeStruct((NPAD, 1), jnp.float32)],
  )(d0, d1, x)


def _tc_layer0(y0, y1, dinv, w0, b0):
  def body(y0_ref, y1_ref, dinv_ref, w_ref, b_ref, h_ref, xs_ref):
    agg = (y0_ref[...] + y1_ref[...]) * dinv_ref[...]
    h = jnp.dot(agg, w_ref[...], preferred_element_type=jnp.float32)
    h = jnp.maximum(h + b_ref[...], 0.0)
    h_ref[...] = h
    xs_ref[...] = h * dinv_ref[...]

  return pl.pallas_call(
      body,
      grid=(NPAD // _R,),
      in_specs=[_row_spec(D_HID), _row_spec(D_HID), _row_spec(1),
                _full_spec((D_HID, D_HID)), _full_spec((1, D_HID))],
      out_specs=[_row_spec(D_HID), _row_spec(D_HID)],
      out_shape=[jax.ShapeDtypeStruct((NPAD, D_HID), jnp.float32),
                 jax.ShapeDtypeStruct((NPAD, D_HID), jnp.float32)],
  )(y0, y1, dinv, w0, b0)


def _tc_layer1(y0, y1, dinv, w1, b1, w2, wi, h1):
  def body(y0_ref, y1_ref, dinv_ref, w1_ref, b1_ref, w2_ref, wi_ref, h1_ref,
           z_ref, skip_ref):
    agg = (y0_ref[...] + y1_ref[...]) * dinv_ref[...]
    h2 = jnp.dot(agg, w1_ref[...], preferred_element_type=jnp.float32)
    h2 = jnp.maximum(h2 + b1_ref[...], 0.0)
    z_ref[...] = jnp.dot(h2 * dinv_ref[...], w2_ref[...],
                         preferred_element_type=jnp.float32)
    skip_ref[...] = jnp.dot(h1_ref[...] + h2, wi_ref[...],
                            preferred_element_type=jnp.float32)

  return pl.pallas_call(
      body,
      grid=(NPAD // _R,),
      in_specs=[_row_spec(D_HID), _row_spec(D_HID), _row_spec(1),
                _full_spec((D_HID, D_HID)), _full_spec((1, D_HID)),
                _full_spec((D_HID, D_OUT)), _full_spec((D_HID, D_OUT)),
                _row_spec(D_HID)],
      out_specs=[_row_spec(D_OUT), _row_spec(D_OUT)],
      out_shape=[jax.ShapeDtypeStruct((NPAD, D_OUT), jnp.float32),
                 jax.ShapeDtypeStruct((NPAD, D_OUT), jnp.float32)],
  )(y0, y1, dinv, w1, b1, w2, wi, h1)


def _tc_final(z0, z1, dinv, skip, b2, bi):
  def body(z0_ref, z1_ref, dinv_ref, skip_ref, b2_ref, bi_ref, out_ref):
    h3 = jnp.maximum((z0_ref[...] + z1_ref[...]) * dinv_ref[...]
                     + b2_ref[...], 0.0)
    out_ref[...] = skip_ref[...] + 2.0 * bi_ref[...] + h3

  return pl.pallas_call(
      body,
      grid=(NPAD // _R,),
      in_specs=[_row_spec(D_OUT), _row_spec(D_OUT), _row_spec(1),
                _row_spec(D_OUT), _full_spec((1, D_OUT)),
                _full_spec((1, D_OUT))],
      out_specs=_row_spec(D_OUT),
      out_shape=jax.ShapeDtypeStruct((NPAD, D_OUT), jnp.float32),
  )(z0, z1, dinv, skip, b2, bi)


def kernel(features, edge_index, W0, b0, W1, b1, W2, b2, Wi, bi):
  src = jnp.pad(edge_index[0], (0, EPAD - E)).reshape(EPAD // SEG_CHUNK,
                                                      SEG_CHUNK)
  dst = jnp.pad(edge_index[1], (0, EPAD - E),
                constant_values=NPAD - 1).reshape(EPAD // SEG_CHUNK,
                                                  SEG_CHUNK)
  ones_c = jnp.ones((DEG_CHUNK,), jnp.float32)
  zeros_1 = jnp.zeros((NPAD,), jnp.float32)

  dg0, dg1 = _deg(dst, ones_c, zeros_1)
  d0 = dg0.reshape(NPAD, 1)
  d1 = dg1.reshape(NPAD, 1)

  x_pad = jnp.pad(features, ((0, NPAD - N), (0, 0)))
  xs0, dinv = _tc_prep(d0, d1, x_pad)
  y0, y1 = _seg128(xs0, src, dst)
  h1, xs1 = _tc_layer0(y0, y1, dinv, W0, b0.reshape(1, D_HID))
  y0, y1 = _seg128(xs1, src, dst)
  z, skip = _tc_layer1(y0, y1, dinv, W1, b1.reshape(1, D_HID), W2, Wi, h1)
  z0, z1 = _seg64(z, src, dst)
  out = _tc_final(z0, z1, dinv, skip, b2.reshape(1, D_OUT),
                  bi.reshape(1, D_OUT))
  return out[:N]


# speedup vs baseline: 1.1701x; 1.1701x over previous
import functools

import jax
import jax.numpy as jnp
from jax import lax
from jax.experimental import pallas as pl
from jax.experimental.pallas import tpu as pltpu
from jax.experimental.pallas import tpu_sc as plsc

N = 10000
E = 320000
D_HID = 128
D_OUT = 64

NC = 2
NS = 16
NTILE = NC * NS
NPAD = 10240
DEG_SLICE = NPAD // NS

EPAD = 327680
EPT = EPAD // NTILE

SEG_CHUNK = 128
SEG_G = EPT // SEG_CHUNK
NBUF = 2

DEG_CHUNK = 128
DEG_G = EPT // DEG_CHUNK

ROWS_PT = NPAD // NS
STG = 128

def _zero16():
  return jnp.zeros((16,), jnp.float32)


def _make_seg_sum(D, linear=False):
  HG = SEG_G // 2
  mesh = plsc.VectorSubcoreMesh(core_axis_name="c", subcore_axis_name="s")
  out_t = (jax.ShapeDtypeStruct((NPAD, D), jnp.float32),
           jax.ShapeDtypeStruct((NPAD, D), jnp.float32))
  scratch = ([pltpu.VMEM_SHARED((NPAD, D), jnp.float32),
              pltpu.VMEM((HG, SEG_CHUNK), jnp.int32),
              pltpu.VMEM((HG, SEG_CHUNK), jnp.int32),
              pltpu.VMEM((16, D), jnp.float32)]
             + [pltpu.VMEM((STG, D), jnp.float32) for _ in range(NBUF)]
             + [pltpu.SemaphoreType.DMA for _ in range(NBUF)])

  params = pltpu.CompilerParams(use_tc_tiling_on_sc=False) if linear else None

  @functools.partial(pl.kernel, out_type=out_t, mesh=mesh,
                     scratch_types=scratch, compiler_params=params)
  def seg(xs, src, dst, out0, out1, acc, srcv, dstv, zbuf,
          r0, r1, m0, m1):
    c = lax.axis_index("c")
    s = lax.axis_index("s")
    w = c * NS + s
    rows = (r0, r1)
    sems = (m0, m1)

    for r in range(16):
      for j in range(D // 16):
        zbuf[r, pl.ds(j * 16, 16)] = _zero16()

    def _zacc(i, carry):
      pltpu.sync_copy(zbuf, acc.at[pl.ds(s * ROWS_PT + i * 16, 16)])
      return carry
    lax.fori_loop(0, ROWS_PT // 16, _zacc, 0)
    plsc.subcore_barrier()

    def _fire(g, j):
      pltpu.async_copy(xs.at[srcv.at[g]], rows[j], sems[j])

    def _wait(g, j):
      pltpu.make_async_copy(xs.at[srcv.at[g]], rows[j], sems[j]).wait()

    for h in range(2):
      pltpu.sync_copy(src.at[pl.ds(w * SEG_G + h * HG, HG)], srcv)
      pltpu.sync_copy(dst.at[pl.ds(w * SEG_G + h * HG, HG)], dstv)

      for j in range(NBUF):
        _fire(j, j)

      def _group(i, carry):
        for j in range(NBUF):
          g = i * NBUF + j
          _wait(g, j)
          pltpu.sync_copy(rows[j], acc.at[dstv.at[g]], add=True)

          @pl.when(g + NBUF < HG)
          def _():
            _fire(g + NBUF, j)
        return carry
      lax.fori_loop(0, HG // NBUF, _group, 0)

    plsc.subcore_barrier()

    for k in range(ROWS_PT // STG):
      base = s * ROWS_PT + k * STG
      pltpu.sync_copy(acc.at[pl.ds(base, STG)], rows[0])

      @pl.when(c == 0)
      def _():
        pltpu.sync_copy(rows[0], out0.at[pl.ds(base, STG)])

      @pl.when(c == 1)
      def _():
        pltpu.sync_copy(rows[0], out1.at[pl.ds(base, STG)])

  return seg


def _make_deg():
  mesh = plsc.VectorSubcoreMesh(core_axis_name="c", subcore_axis_name="s")
  out_t = (jax.ShapeDtypeStruct((NPAD,), jnp.float32),
           jax.ShapeDtypeStruct((NPAD,), jnp.float32))
  scratch = ([pltpu.VMEM_SHARED((NPAD,), jnp.float32),
              pltpu.VMEM((DEG_G, DEG_CHUNK), jnp.int32),
              pltpu.VMEM((DEG_CHUNK,), jnp.float32)]
             + [pltpu.SemaphoreType.DMA for _ in range(4)])

  @functools.partial(pl.kernel, out_type=out_t, mesh=mesh,
                     scratch_types=scratch)
  def deg(dst, ones, zeros, out0, out1, acc, dstv, ones_v, m0, m1, m2, m3):
    c = lax.axis_index("c")
    s = lax.axis_index("s")
    w = c * NS + s
    sems = (m0, m1, m2, m3)

    pltpu.sync_copy(ones, ones_v)
    pltpu.sync_copy(dst.at[pl.ds(w * DEG_G, DEG_G)], dstv)
    pltpu.sync_copy(zeros.at[pl.ds(s * DEG_SLICE, DEG_SLICE)],
                    acc.at[pl.ds(s * DEG_SLICE, DEG_SLICE)])
    plsc.subcore_barrier()

    def _fire(g, j):
      pltpu.async_copy(ones_v, acc.at[dstv.at[g]], sems[j], add=True)

    def _wait(g, j):
      pltpu.make_async_copy(ones_v, acc.at[dstv.at[g]], sems[j]).wait()

    for j in range(4):
      _fire(j, j)

    def _group(i, carry):
      for j in range(4):
        g = i * 4 + j
        _wait(g, j)

        @pl.when(g + 4 < DEG_G)
        def _():
          _fire(g + 4, j)
      return carry
    lax.fori_loop(0, DEG_G // 4, _group, 0)

    plsc.subcore_barrier()

    @pl.when(c == 0)
    def _():
      pltpu.sync_copy(acc.at[pl.ds(s * DEG_SLICE, DEG_SLICE)],
                      out0.at[pl.ds(s * DEG_SLICE, DEG_SLICE)])

    @pl.when(c == 1)
    def _():
      pltpu.sync_copy(acc.at[pl.ds(s * DEG_SLICE, DEG_SLICE)],
                      out1.at[pl.ds(s * DEG_SLICE, DEG_SLICE)])

  return deg


_seg128 = _make_seg_sum(D_HID)
_seg64 = _make_seg_sum(D_OUT, linear=True)
_deg = _make_deg()

_R = 2048


def _row_spec(d):
  return pl.BlockSpec((_R, d), lambda i: (i, 0))


def _full_spec(shape):
  return pl.BlockSpec(shape, lambda i: (0, 0))


def _tc_prep(d0, d1, x):
  def body(d0_ref, d1_ref, x_ref, xs_ref, dinv_ref):
    deg = jnp.maximum(d0_ref[...] + d1_ref[...], 1.0)
    dinv = lax.rsqrt(deg)
    dinv_ref[...] = dinv
    xs_ref[...] = x_ref[...] * dinv

  return pl.pallas_call(
      body,
      grid=(NPAD // _R,),
      in_specs=[_row_spec(1), _row_spec(1), _row_spec(D_HID)],
      out_specs=[_row_spec(D_HID), _row_spec(1)],
      out_shape=[jax.ShapeDtypeStruct((NPAD, D_HID), jnp.float32),
                 jax.ShapeDtypeStruct((NPAD, 1), jnp.float32)],
  )(d0, d1, x)


def _tc_layer0(y0, y1, dinv, w0, b0):
  def body(y0_ref, y1_ref, dinv_ref, w_ref, b_ref, h_ref, xs_ref):
    agg = (y0_ref[...] + y1_ref[...]) * dinv_ref[...]
    h = jnp.dot(agg, w_ref[...], preferred_element_type=jnp.float32)
    h = jnp.maximum(h + b_ref[...], 0.0)
    h_ref[...] = h
    xs_ref[...] = h * dinv_ref[...]

  return pl.pallas_call(
      body,
      grid=(NPAD // _R,),
      in_specs=[_row_spec(D_HID), _row_spec(D_HID), _row_spec(1),
                _full_spec((D_HID, D_HID)), _full_spec((1, D_HID))],
      out_specs=[_row_spec(D_HID), _row_spec(D_HID)],
      out_shape=[jax.ShapeDtypeStruct((NPAD, D_HID), jnp.float32),
                 jax.ShapeDtypeStruct((NPAD, D_HID), jnp.float32)],
  )(y0, y1, dinv, w0, b0)


def _tc_layer1(y0, y1, dinv, w1, b1, w2, wi, h1):
  def body(y0_ref, y1_ref, dinv_ref, w1_ref, b1_ref, w2_ref, wi_ref, h1_ref,
           z_ref, skip_ref):
    agg = (y0_ref[...] + y1_ref[...]) * dinv_ref[...]
    h2 = jnp.dot(agg, w1_ref[...], preferred_element_type=jnp.float32)
    h2 = jnp.maximum(h2 + b1_ref[...], 0.0)
    z_ref[...] = jnp.dot(h2 * dinv_ref[...], w2_ref[...],
                         preferred_element_type=jnp.float32)
    skip_ref[...] = jnp.dot(h1_ref[...] + h2, wi_ref[...],
                            preferred_element_type=jnp.float32)

  return pl.pallas_call(
      body,
      grid=(NPAD // _R,),
      in_specs=[_row_spec(D_HID), _row_spec(D_HID), _row_spec(1),
                _full_spec((D_HID, D_HID)), _full_spec((1, D_HID)),
                _full_spec((D_HID, D_OUT)), _full_spec((D_HID, D_OUT)),
                _row_spec(D_HID)],
      out_specs=[_row_spec(D_OUT), _row_spec(D_OUT)],
      out_shape=[jax.ShapeDtypeStruct((NPAD, D_OUT), jnp.float32),
                 jax.ShapeDtypeStruct((NPAD, D_OUT), jnp.float32)],
  )(y0, y1, dinv, w1, b1, w2, wi, h1)


def _tc_final(z0, z1, dinv, skip, b2, bi):
  def body(z0_ref, z1_ref, dinv_ref, skip_ref, b2_ref, bi_ref, out_ref):
    h3 = jnp.maximum((z0_ref[...] + z1_ref[...]) * dinv_ref[...]
                     + b2_ref[...], 0.0)
    out_ref[...] = skip_ref[...] + 2.0 * bi_ref[...] + h3

  return pl.pallas_call(
      body,
      grid=(NPAD // _R,),
      in_specs=[_row_spec(D_OUT), _row_spec(D_OUT), _row_spec(1),
                _row_spec(D_OUT), _full_spec((1, D_OUT)),
                _full_spec((1, D_OUT))],
      out_specs=_row_spec(D_OUT),
      out_shape=jax.ShapeDtypeStruct((NPAD, D_OUT), jnp.float32),
  )(z0, z1, dinv, skip, b2, bi)


def kernel(features, edge_index, W0, b0, W1, b1, W2, b2, Wi, bi):
  src = jnp.pad(edge_index[0], (0, EPAD - E)).reshape(EPAD // SEG_CHUNK,
                                                      SEG_CHUNK)
  pad_dst = N + (jnp.arange(EPAD - E, dtype=jnp.int32) % (NPAD - N))
  dst = jnp.concatenate([edge_index[1], pad_dst]).reshape(EPAD // SEG_CHUNK,
                                                          SEG_CHUNK)
  ones_c = jnp.ones((DEG_CHUNK,), jnp.float32)
  zeros_1 = jnp.zeros((NPAD,), jnp.float32)

  dg0, dg1 = _deg(dst, ones_c, zeros_1)
  d0 = dg0.reshape(NPAD, 1)
  d1 = dg1.reshape(NPAD, 1)

  x_pad = jnp.pad(features, ((0, NPAD - N), (0, 0)))
  xs0, dinv = _tc_prep(d0, d1, x_pad)
  y0, y1 = _seg128(xs0, src, dst)
  h1, xs1 = _tc_layer0(y0, y1, dinv, W0, b0.reshape(1, D_HID))
  y0, y1 = _seg128(xs1, src, dst)
  z, skip = _tc_layer1(y0, y1, dinv, W1, b1.reshape(1, D_HID), W2, Wi, h1)
  z0, z1 = _seg64(z, src, dst)
  out = _tc_final(z0, z1, dinv, skip, b2.reshape(1, D_OUT),
                  bi.reshape(1, D_OUT))
  return out[:N]


# speedup vs baseline: 3.4768x; 2.9715x over previous
import functools

import jax
import jax.numpy as jnp
from jax import lax
from jax.experimental import pallas as pl
from jax.experimental.pallas import tpu as pltpu
from jax.experimental.pallas import tpu_sc as plsc

N = 10000
E = 320000
D_HID = 128
D_OUT = 64

NC = 2
NS = 16
NTILE = NC * NS
NPAD = 10240
DEG_SLICE = NPAD // NS

EPT = E // NTILE

SEG_CHUNK = 125
SEG_G = EPT // SEG_CHUNK
NBUF = 2

DEG_CHUNK = 125
DEG_G = EPT // DEG_CHUNK

ROWS_PT = NPAD // NS
STG = 128

def _zero16():
  return jnp.zeros((16,), jnp.float32)


def _make_seg_sum(D, linear=False):
  HG = SEG_G // 2
  mesh = plsc.VectorSubcoreMesh(core_axis_name="c", subcore_axis_name="s")
  out_t = (jax.ShapeDtypeStruct((NPAD, D), jnp.float32),
           jax.ShapeDtypeStruct((NPAD, D), jnp.float32))
  scratch = ([pltpu.VMEM_SHARED((NPAD, D), jnp.float32),
              pltpu.VMEM((HG, SEG_CHUNK), jnp.int32),
              pltpu.VMEM((HG, SEG_CHUNK), jnp.int32),
              pltpu.VMEM((16, D), jnp.float32)]
             + [pltpu.VMEM((STG, D), jnp.float32) for _ in range(NBUF)]
             + [pltpu.SemaphoreType.DMA for _ in range(NBUF)])

  params = pltpu.CompilerParams(use_tc_tiling_on_sc=False) if linear else None

  @functools.partial(pl.kernel, out_type=out_t, mesh=mesh,
                     scratch_types=scratch, compiler_params=params)
  def seg(xs, src, dst, out0, out1, acc, srcv, dstv, zbuf,
          r0, r1, m0, m1):
    c = lax.axis_index("c")
    s = lax.axis_index("s")
    w = c * NS + s
    rows = (r0, r1)
    sems = (m0, m1)

    for r in range(16):
      for j in range(D // 16):
        zbuf[r, pl.ds(j * 16, 16)] = _zero16()

    def _zacc(i, carry):
      pltpu.sync_copy(zbuf, acc.at[pl.ds(s * ROWS_PT + i * 16, 16)])
      return carry
    lax.fori_loop(0, ROWS_PT // 16, _zacc, 0)
    plsc.subcore_barrier()

    def _fire(g, j):
      pltpu.async_copy(xs.at[srcv.at[g]], rows[j].at[pl.ds(0, SEG_CHUNK)],
                       sems[j])

    def _wait(g, j):
      pltpu.make_async_copy(xs.at[srcv.at[g]],
                            rows[j].at[pl.ds(0, SEG_CHUNK)], sems[j]).wait()

    for h in range(2):
      pltpu.sync_copy(src.at[pl.ds(w * SEG_G + h * HG, HG)], srcv)
      pltpu.sync_copy(dst.at[pl.ds(w * SEG_G + h * HG, HG)], dstv)

      for j in range(NBUF):
        _fire(j, j)

      def _group(i, carry):
        for j in range(NBUF):
          g = i * NBUF + j
          _wait(g, j)
          pltpu.sync_copy(rows[j].at[pl.ds(0, SEG_CHUNK)],
                          acc.at[dstv.at[g]], add=True)

          @pl.when(g + NBUF < HG)
          def _():
            _fire(g + NBUF, j)
        return carry
      lax.fori_loop(0, HG // NBUF, _group, 0)

    plsc.subcore_barrier()

    for k in range(ROWS_PT // STG):
      base = s * ROWS_PT + k * STG
      pltpu.sync_copy(acc.at[pl.ds(base, STG)], rows[0])

      @pl.when(c == 0)
      def _():
        pltpu.sync_copy(rows[0], out0.at[pl.ds(base, STG)])

      @pl.when(c == 1)
      def _():
        pltpu.sync_copy(rows[0], out1.at[pl.ds(base, STG)])

  return seg


def _make_deg():
  mesh = plsc.VectorSubcoreMesh(core_axis_name="c", subcore_axis_name="s")
  out_t = (jax.ShapeDtypeStruct((NPAD,), jnp.float32),
           jax.ShapeDtypeStruct((NPAD,), jnp.float32))
  scratch = ([pltpu.VMEM_SHARED((NPAD,), jnp.float32),
              pltpu.VMEM((DEG_G, DEG_CHUNK), jnp.int32),
              pltpu.VMEM((DEG_CHUNK,), jnp.float32)]
             + [pltpu.SemaphoreType.DMA for _ in range(4)])

  @functools.partial(pl.kernel, out_type=out_t, mesh=mesh,
                     scratch_types=scratch)
  def deg(dst, ones, zeros, out0, out1, acc, dstv, ones_v, m0, m1, m2, m3):
    c = lax.axis_index("c")
    s = lax.axis_index("s")
    w = c * NS + s
    sems = (m0, m1, m2, m3)

    pltpu.sync_copy(ones, ones_v)
    pltpu.sync_copy(dst.at[pl.ds(w * DEG_G, DEG_G)], dstv)
    pltpu.sync_copy(zeros.at[pl.ds(s * DEG_SLICE, DEG_SLICE)],
                    acc.at[pl.ds(s * DEG_SLICE, DEG_SLICE)])
    plsc.subcore_barrier()

    def _fire(g, j):
      pltpu.async_copy(ones_v, acc.at[dstv.at[g]], sems[j], add=True)

    def _wait(g, j):
      pltpu.make_async_copy(ones_v, acc.at[dstv.at[g]], sems[j]).wait()

    for j in range(4):
      _fire(j, j)

    def _group(i, carry):
      for j in range(4):
        g = i * 4 + j
        _wait(g, j)

        @pl.when(g + 4 < DEG_G)
        def _():
          _fire(g + 4, j)
      return carry
    lax.fori_loop(0, DEG_G // 4, _group, 0)

    plsc.subcore_barrier()

    @pl.when(c == 0)
    def _():
      pltpu.sync_copy(acc.at[pl.ds(s * DEG_SLICE, DEG_SLICE)],
                      out0.at[pl.ds(s * DEG_SLICE, DEG_SLICE)])

    @pl.when(c == 1)
    def _():
      pltpu.sync_copy(acc.at[pl.ds(s * DEG_SLICE, DEG_SLICE)],
                      out1.at[pl.ds(s * DEG_SLICE, DEG_SLICE)])

  return deg


_seg128 = _make_seg_sum(D_HID)
_seg64 = _make_seg_sum(D_OUT, linear=True)
_deg = _make_deg()

_R = 2048


def _row_spec(d):
  return pl.BlockSpec((_R, d), lambda i: (i, 0))


def _full_spec(shape):
  return pl.BlockSpec(shape, lambda i: (0, 0))


def _tc_prep(d0, d1, x):
  def body(d0_ref, d1_ref, x_ref, xs_ref, dinv_ref):
    deg = jnp.maximum(d0_ref[...] + d1_ref[...], 1.0)
    dinv = lax.rsqrt(deg)
    dinv_ref[...] = dinv
    xs_ref[...] = x_ref[...] * dinv

  return pl.pallas_call(
      body,
      grid=(NPAD // _R,),
      in_specs=[_row_spec(1), _row_spec(1), _row_spec(D_HID)],
      out_specs=[_row_spec(D_HID), _row_spec(1)],
      out_shape=[jax.ShapeDtypeStruct((NPAD, D_HID), jnp.float32),
                 jax.ShapeDtypeStruct((NPAD, 1), jnp.float32)],
  )(d0, d1, x)


def _tc_layer0(y0, y1, dinv, w0, b0):
  def body(y0_ref, y1_ref, dinv_ref, w_ref, b_ref, h_ref, xs_ref):
    agg = (y0_ref[...] + y1_ref[...]) * dinv_ref[...]
    h = jnp.dot(agg, w_ref[...], preferred_element_type=jnp.float32)
    h = jnp.maximum(h + b_ref[...], 0.0)
    h_ref[...] = h
    xs_ref[...] = h * dinv_ref[...]

  return pl.pallas_call(
      body,
      grid=(NPAD // _R,),
      in_specs=[_row_spec(D_HID), _row_spec(D_HID), _row_spec(1),
                _full_spec((D_HID, D_HID)), _full_spec((1, D_HID))],
      out_specs=[_row_spec(D_HID), _row_spec(D_HID)],
      out_shape=[jax.ShapeDtypeStruct((NPAD, D_HID), jnp.float32),
                 jax.ShapeDtypeStruct((NPAD, D_HID), jnp.float32)],
  )(y0, y1, dinv, w0, b0)


def _tc_layer1(y0, y1, dinv, w1, b1, w2, wi, h1):
  def body(y0_ref, y1_ref, dinv_ref, w1_ref, b1_ref, w2_ref, wi_ref, h1_ref,
           z_ref, skip_ref):
    agg = (y0_ref[...] + y1_ref[...]) * dinv_ref[...]
    h2 = jnp.dot(agg, w1_ref[...], preferred_element_type=jnp.float32)
    h2 = jnp.maximum(h2 + b1_ref[...], 0.0)
    z_ref[...] = jnp.dot(h2 * dinv_ref[...], w2_ref[...],
                         preferred_element_type=jnp.float32)
    skip_ref[...] = jnp.dot(h1_ref[...] + h2, wi_ref[...],
                            preferred_element_type=jnp.float32)

  return pl.pallas_call(
      body,
      grid=(NPAD // _R,),
      in_specs=[_row_spec(D_HID), _row_spec(D_HID), _row_spec(1),
                _full_spec((D_HID, D_HID)), _full_spec((1, D_HID)),
                _full_spec((D_HID, D_OUT)), _full_spec((D_HID, D_OUT)),
                _row_spec(D_HID)],
      out_specs=[_row_spec(D_OUT), _row_spec(D_OUT)],
      out_shape=[jax.ShapeDtypeStruct((NPAD, D_OUT), jnp.float32),
                 jax.ShapeDtypeStruct((NPAD, D_OUT), jnp.float32)],
  )(y0, y1, dinv, w1, b1, w2, wi, h1)


def _tc_final(z0, z1, dinv, skip, b2, bi):
  def body(z0_ref, z1_ref, dinv_ref, skip_ref, b2_ref, bi_ref, out_ref):
    h3 = jnp.maximum((z0_ref[...] + z1_ref[...]) * dinv_ref[...]
                     + b2_ref[...], 0.0)
    out_ref[...] = skip_ref[...] + 2.0 * bi_ref[...] + h3

  return pl.pallas_call(
      body,
      grid=(NPAD // _R,),
      in_specs=[_row_spec(D_OUT), _row_spec(D_OUT), _row_spec(1),
                _row_spec(D_OUT), _full_spec((1, D_OUT)),
                _full_spec((1, D_OUT))],
      out_specs=_row_spec(D_OUT),
      out_shape=jax.ShapeDtypeStruct((NPAD, D_OUT), jnp.float32),
  )(z0, z1, dinv, skip, b2, bi)


def kernel(features, edge_index, W0, b0, W1, b1, W2, b2, Wi, bi):
  src = edge_index[0].reshape(E // SEG_CHUNK, SEG_CHUNK)
  dst = edge_index[1].reshape(E // SEG_CHUNK, SEG_CHUNK)
  ones_c = jnp.ones((DEG_CHUNK,), jnp.float32)
  zeros_1 = jnp.zeros((NPAD,), jnp.float32)

  dg0, dg1 = _deg(dst, ones_c, zeros_1)
  d0 = dg0.reshape(NPAD, 1)
  d1 = dg1.reshape(NPAD, 1)

  x_pad = jnp.pad(features, ((0, NPAD - N), (0, 0)))
  xs0, dinv = _tc_prep(d0, d1, x_pad)
  y0, y1 = _seg128(xs0, src, dst)
  h1, xs1 = _tc_layer0(y0, y1, dinv, W0, b0.reshape(1, D_HID))
  y0, y1 = _seg128(xs1, src, dst)
  z, skip = _tc_layer1(y0, y1, dinv, W1, b1.reshape(1, D_HID), W2, Wi, h1)
  z0, z1 = _seg64(z, src, dst)
  out = _tc_final(z0, z1, dinv, skip, b2.reshape(1, D_OUT),
                  bi.reshape(1, D_OUT))
  return out[:N]


# speedup vs baseline: 3.6115x; 1.0387x over previous
import functools

import jax
import jax.numpy as jnp
from jax import lax
from jax.experimental import pallas as pl
from jax.experimental.pallas import tpu as pltpu
from jax.experimental.pallas import tpu_sc as plsc

N = 10000
E = 320000
D_HID = 128
D_OUT = 64

NC = 2
NS = 16
NTILE = NC * NS
NPAD = 10240
DEG_SLICE = NPAD // NS

EPT = E // NTILE

SEG_CHUNK = 125
SEG_G = EPT // SEG_CHUNK
NBUF = 2

DEG_CHUNK = 125
DEG_G = EPT // DEG_CHUNK

ROWS_PT = NPAD // NS
STG = 128

def _zero16():
  return jnp.zeros((16,), jnp.float32)


def _make_seg_sum(D, linear=False):
  HG = SEG_G // 2
  mesh = plsc.VectorSubcoreMesh(core_axis_name="c", subcore_axis_name="s")
  out_t = (jax.ShapeDtypeStruct((NPAD, D), jnp.float32),
           jax.ShapeDtypeStruct((NPAD, D), jnp.float32))
  scratch = ([pltpu.VMEM_SHARED((NPAD, D), jnp.float32),
              pltpu.VMEM((HG, SEG_CHUNK), jnp.int32),
              pltpu.VMEM((HG, SEG_CHUNK), jnp.int32),
              pltpu.VMEM((16, D), jnp.float32)]
             + [pltpu.VMEM((STG, D), jnp.float32) for _ in range(NBUF)]
             + [pltpu.SemaphoreType.DMA for _ in range(NBUF)])

  params = pltpu.CompilerParams(use_tc_tiling_on_sc=False) if linear else None

  @functools.partial(pl.kernel, out_type=out_t, mesh=mesh,
                     scratch_types=scratch, compiler_params=params)
  def seg(xs, src, dst, out0, out1, acc, srcv, dstv, zbuf,
          r0, r1, m0, m1):
    c = lax.axis_index("c")
    s = lax.axis_index("s")
    w = c * NS + s
    rows = (r0, r1)
    sems = (m0, m1)

    pltpu.sync_copy(src.at[pl.ds(w * SEG_G, HG)], srcv)
    pltpu.sync_copy(dst.at[pl.ds(w * SEG_G, HG)], dstv)

    def _fire(g, j):
      pltpu.async_copy(xs.at[srcv.at[g]], rows[j].at[pl.ds(0, SEG_CHUNK)],
                       sems[j])

    def _wait(g, j):
      pltpu.make_async_copy(xs.at[srcv.at[g]],
                            rows[j].at[pl.ds(0, SEG_CHUNK)], sems[j]).wait()

    for j in range(NBUF):
      _fire(j, j)

    for r in range(16):
      for j in range(D // 16):
        zbuf[r, pl.ds(j * 16, 16)] = _zero16()

    def _zacc(i, carry):
      pltpu.sync_copy(zbuf, acc.at[pl.ds(s * ROWS_PT + i * 16, 16)])
      return carry
    lax.fori_loop(0, ROWS_PT // 16, _zacc, 0)
    plsc.subcore_barrier()

    for h in range(2):
      if h > 0:
        pltpu.sync_copy(src.at[pl.ds(w * SEG_G + h * HG, HG)], srcv)
        pltpu.sync_copy(dst.at[pl.ds(w * SEG_G + h * HG, HG)], dstv)
        for j in range(NBUF):
          _fire(j, j)

      def _group(i, carry):
        for j in range(NBUF):
          g = i * NBUF + j
          _wait(g, j)
          pltpu.sync_copy(rows[j].at[pl.ds(0, SEG_CHUNK)],
                          acc.at[dstv.at[g]], add=True)

          @pl.when(g + NBUF < HG)
          def _():
            _fire(g + NBUF, j)
        return carry
      lax.fori_loop(0, HG // NBUF, _group, 0)

    plsc.subcore_barrier()

    def _exp_fire(k, j):
      @pl.when(c == 0)
      def _():
        pltpu.async_copy(rows[j], out0.at[pl.ds(s * ROWS_PT + k * STG, STG)],
                         sems[j])

      @pl.when(c == 1)
      def _():
        pltpu.async_copy(rows[j], out1.at[pl.ds(s * ROWS_PT + k * STG, STG)],
                         sems[j])

    def _exp_wait(k, j):
      @pl.when(c == 0)
      def _():
        pltpu.make_async_copy(
            rows[j], out0.at[pl.ds(s * ROWS_PT + k * STG, STG)],
            sems[j]).wait()

      @pl.when(c == 1)
      def _():
        pltpu.make_async_copy(
            rows[j], out1.at[pl.ds(s * ROWS_PT + k * STG, STG)],
            sems[j]).wait()

    for k in range(ROWS_PT // STG):
      j = k % 2
      if k >= 2:
        _exp_wait(k - 2, j)
      pltpu.sync_copy(acc.at[pl.ds(s * ROWS_PT + k * STG, STG)], rows[j])
      _exp_fire(k, j)
    _exp_wait(3, 1)
    _exp_wait(4, 0)

  return seg


def _make_deg():
  mesh = plsc.VectorSubcoreMesh(core_axis_name="c", subcore_axis_name="s")
  out_t = (jax.ShapeDtypeStruct((NPAD,), jnp.float32),
           jax.ShapeDtypeStruct((NPAD,), jnp.float32))
  scratch = ([pltpu.VMEM_SHARED((NPAD,), jnp.float32),
              pltpu.VMEM((DEG_G, DEG_CHUNK), jnp.int32),
              pltpu.VMEM((DEG_CHUNK,), jnp.float32)]
             + [pltpu.SemaphoreType.DMA for _ in range(4)])

  @functools.partial(pl.kernel, out_type=out_t, mesh=mesh,
                     scratch_types=scratch)
  def deg(dst, ones, zeros, out0, out1, acc, dstv, ones_v, m0, m1, m2, m3):
    c = lax.axis_index("c")
    s = lax.axis_index("s")
    w = c * NS + s
    sems = (m0, m1, m2, m3)

    pltpu.sync_copy(ones, ones_v)
    pltpu.sync_copy(dst.at[pl.ds(w * DEG_G, DEG_G)], dstv)
    pltpu.sync_copy(zeros.at[pl.ds(s * DEG_SLICE, DEG_SLICE)],
                    acc.at[pl.ds(s * DEG_SLICE, DEG_SLICE)])
    plsc.subcore_barrier()

    def _fire(g, j):
      pltpu.async_copy(ones_v, acc.at[dstv.at[g]], sems[j], add=True)

    def _wait(g, j):
      pltpu.make_async_copy(ones_v, acc.at[dstv.at[g]], sems[j]).wait()

    for j in range(4):
      _fire(j, j)

    def _group(i, carry):
      for j in range(4):
        g = i * 4 + j
        _wait(g, j)

        @pl.when(g + 4 < DEG_G)
        def _():
          _fire(g + 4, j)
      return carry
    lax.fori_loop(0, DEG_G // 4, _group, 0)

    plsc.subcore_barrier()

    @pl.when(c == 0)
    def _():
      pltpu.sync_copy(acc.at[pl.ds(s * DEG_SLICE, DEG_SLICE)],
                      out0.at[pl.ds(s * DEG_SLICE, DEG_SLICE)])

    @pl.when(c == 1)
    def _():
      pltpu.sync_copy(acc.at[pl.ds(s * DEG_SLICE, DEG_SLICE)],
                      out1.at[pl.ds(s * DEG_SLICE, DEG_SLICE)])

  return deg


_seg128 = _make_seg_sum(D_HID)
_seg64 = _make_seg_sum(D_OUT, linear=True)
_deg = _make_deg()

_R = 5120


def _row_spec(d):
  return pl.BlockSpec((_R, d), lambda i: (i, 0))


def _full_spec(shape):
  return pl.BlockSpec(shape, lambda i: (0, 0))


def _tc_prep(d0, d1, x):
  def body(d0_ref, d1_ref, x_ref, xs_ref, dinv_ref):
    deg = jnp.maximum(d0_ref[...] + d1_ref[...], 1.0)
    dinv = lax.rsqrt(deg)
    dinv_ref[...] = dinv
    xs_ref[...] = x_ref[...] * dinv

  return pl.pallas_call(
      body,
      grid=(NPAD // _R,),
      in_specs=[_row_spec(1), _row_spec(1), _row_spec(D_HID)],
      out_specs=[_row_spec(D_HID), _row_spec(1)],
      out_shape=[jax.ShapeDtypeStruct((NPAD, D_HID), jnp.float32),
                 jax.ShapeDtypeStruct((NPAD, 1), jnp.float32)],
  )(d0, d1, x)


def _tc_layer0(y0, y1, dinv, w0, b0):
  def body(y0_ref, y1_ref, dinv_ref, w_ref, b_ref, h_ref, xs_ref):
    agg = (y0_ref[...] + y1_ref[...]) * dinv_ref[...]
    h = jnp.dot(agg, w_ref[...], preferred_element_type=jnp.float32)
    h = jnp.maximum(h + b_ref[...], 0.0)
    h_ref[...] = h
    xs_ref[...] = h * dinv_ref[...]

  return pl.pallas_call(
      body,
      grid=(NPAD // _R,),
      in_specs=[_row_spec(D_HID), _row_spec(D_HID), _row_spec(1),
                _full_spec((D_HID, D_HID)), _full_spec((1, D_HID))],
      out_specs=[_row_spec(D_HID), _row_spec(D_HID)],
      out_shape=[jax.ShapeDtypeStruct((NPAD, D_HID), jnp.float32),
                 jax.ShapeDtypeStruct((NPAD, D_HID), jnp.float32)],
  )(y0, y1, dinv, w0, b0)


def _tc_layer1(y0, y1, dinv, w1, b1, w2, wi, h1):
  def body(y0_ref, y1_ref, dinv_ref, w1_ref, b1_ref, w2_ref, wi_ref, h1_ref,
           z_ref, skip_ref):
    agg = (y0_ref[...] + y1_ref[...]) * dinv_ref[...]
    h2 = jnp.dot(agg, w1_ref[...], preferred_element_type=jnp.float32)
    h2 = jnp.maximum(h2 + b1_ref[...], 0.0)
    z_ref[...] = jnp.dot(h2 * dinv_ref[...], w2_ref[...],
                         preferred_element_type=jnp.float32)
    skip_ref[...] = jnp.dot(h1_ref[...] + h2, wi_ref[...],
                            preferred_element_type=jnp.float32)

  return pl.pallas_call(
      body,
      grid=(NPAD // _R,),
      in_specs=[_row_spec(D_HID), _row_spec(D_HID), _row_spec(1),
                _full_spec((D_HID, D_HID)), _full_spec((1, D_HID)),
                _full_spec((D_HID, D_OUT)), _full_spec((D_HID, D_OUT)),
                _row_spec(D_HID)],
      out_specs=[_row_spec(D_OUT), _row_spec(D_OUT)],
      out_shape=[jax.ShapeDtypeStruct((NPAD, D_OUT), jnp.float32),
                 jax.ShapeDtypeStruct((NPAD, D_OUT), jnp.float32)],
  )(y0, y1, dinv, w1, b1, w2, wi, h1)


def _tc_final(z0, z1, dinv, skip, b2, bi):
  def body(z0_ref, z1_ref, dinv_ref, skip_ref, b2_ref, bi_ref, out_ref):
    h3 = jnp.maximum((z0_ref[...] + z1_ref[...]) * dinv_ref[...]
                     + b2_ref[...], 0.0)
    out_ref[...] = skip_ref[...] + 2.0 * bi_ref[...] + h3

  return pl.pallas_call(
      body,
      grid=(NPAD // _R,),
      in_specs=[_row_spec(D_OUT), _row_spec(D_OUT), _row_spec(1),
                _row_spec(D_OUT), _full_spec((1, D_OUT)),
                _full_spec((1, D_OUT))],
      out_specs=_row_spec(D_OUT),
      out_shape=jax.ShapeDtypeStruct((NPAD, D_OUT), jnp.float32),
  )(z0, z1, dinv, skip, b2, bi)


def kernel(features, edge_index, W0, b0, W1, b1, W2, b2, Wi, bi):
  src = edge_index[0].reshape(E // SEG_CHUNK, SEG_CHUNK)
  dst = edge_index[1].reshape(E // SEG_CHUNK, SEG_CHUNK)
  ones_c = jnp.ones((DEG_CHUNK,), jnp.float32)
  zeros_1 = jnp.zeros((NPAD,), jnp.float32)

  dg0, dg1 = _deg(dst, ones_c, zeros_1)
  d0 = dg0.reshape(NPAD, 1)
  d1 = dg1.reshape(NPAD, 1)

  x_pad = jnp.pad(features, ((0, NPAD - N), (0, 0)))
  xs0, dinv = _tc_prep(d0, d1, x_pad)
  y0, y1 = _seg128(xs0, src, dst)
  h1, xs1 = _tc_layer0(y0, y1, dinv, W0, b0.reshape(1, D_HID))
  y0, y1 = _seg128(xs1, src, dst)
  z, skip = _tc_layer1(y0, y1, dinv, W1, b1.reshape(1, D_HID), W2, Wi, h1)
  z0, z1 = _seg64(z, src, dst)
  out = _tc_final(z0, z1, dinv, skip, b2.reshape(1, D_OUT),
                  bi.reshape(1, D_OUT))
  return out[:N]
